# baseline (speedup 1.0000x reference)
import functools

import jax
import jax.numpy as jnp
from jax import lax
from jax.experimental import pallas as pl
from jax.experimental.pallas import tpu as pltpu
from jax.experimental.pallas import tpu_sc as plsc

D = 32
R = 5
RP = 16
LANES = 16
NUM_WORKERS = 32
CHUNK = 512


def _make_sc_lookup(n_idx: int):
  per_w = n_idx // NUM_WORKERS
  n_chunks = per_w // CHUNK
  assert n_chunks % 2 == 0 and n_chunks >= 4
  mesh = plsc.VectorSubcoreMesh(core_axis_name="c", subcore_axis_name="s")

  @functools.partial(
      pl.kernel,
      out_type=jax.ShapeDtypeStruct((n_idx, D), jnp.float32),
      mesh=mesh,
      scratch_types=[
          [pltpu.VMEM((CHUNK,), jnp.int32) for _ in range(2)],
          [pltpu.VMEM((CHUNK, D), jnp.float32) for _ in range(2)],
          [pltpu.VMEM((CHUNK, RP), jnp.float32) for _ in range(2)],
          [pltpu.VMEM((CHUNK, D), jnp.float32) for _ in range(2)],
          pltpu.VMEM((R, D), jnp.float32),
          [pltpu.SemaphoreType.DMA for _ in range(2)],
          [pltpu.SemaphoreType.DMA for _ in range(2)],
          [pltpu.SemaphoreType.DMA for _ in range(2)],
      ],
      compiler_params=pltpu.CompilerParams(use_tc_tiling_on_sc=False),
  )
  def lookup(idx_hbm, w_hbm, a_hbm, b_hbm, out_hbm,
             idx_v, w_v, a_v, o_v, b_v, sem_w, sem_a, sem_o):
    wid = lax.axis_index("s") * 2 + lax.axis_index("c")
    base = wid * per_w

    pltpu.sync_copy(b_hbm, b_v)
    b_lo = [b_v[r, pl.ds(0, LANES)] for r in range(R)]
    b_hi = [b_v[r, pl.ds(LANES, LANES)] for r in range(R)]
    lane_r = [jnp.full((LANES, 1), r, jnp.int32) for r in range(R)]
    gdn = lax.GatherDimensionNumbers(
        offset_dims=(), collapsed_slice_dims=(0,), start_index_map=(0,))

    def splat(vec, idx):
      return lax.gather(vec, idx, gdn, (1,),
                        mode=lax.GatherScatterMode.PROMISE_IN_BOUNDS)

    def fetch(c, b):
      pltpu.sync_copy(idx_hbm.at[pl.ds(base + c * CHUNK, CHUNK)], idx_v[b])
      pltpu.async_copy(w_hbm.at[idx_v[b]], w_v[b], sem_w[b])
      pltpu.async_copy(a_hbm.at[idx_v[b]], a_v[b], sem_a[b])

    def wait_gathers(b):
      pltpu.make_async_copy(w_hbm.at[pl.ds(0, CHUNK)], w_v[b], sem_w[b]).wait()
      pltpu.make_async_copy(a_hbm.at[pl.ds(0, CHUNK)], a_v[b], sem_a[b]).wait()

    def wait_out(b):
      pltpu.make_async_copy(
          o_v[b], out_hbm.at[pl.ds(0, CHUNK)], sem_o[b]).wait()

    fetch(0, 0)

    def pair_body(gp, carry):
      for b in (0, 1):
        c = gp * 2 + b

        @pl.when(c + 1 < n_chunks)
        def _():
          fetch(c + 1, 1 - b)

        wait_gathers(b)

        @pl.when(c >= 2)
        def _():
          wait_out(b)

        def row_body(i, carry2):
          av = a_v[b][i, pl.ds(0, LANES)]
          lo = w_v[b][i, pl.ds(0, LANES)]
          hi = w_v[b][i, pl.ds(LANES, LANES)]
          for r in range(R):
            a_r = splat(av, lane_r[r])
            lo = lo + a_r * b_lo[r]
            hi = hi + a_r * b_hi[r]
          o_v[b][i, pl.ds(0, LANES)] = lo
          o_v[b][i, pl.ds(LANES, LANES)] = hi
          return carry2

        lax.fori_loop(0, CHUNK, row_body, 0, unroll=4)
        pltpu.async_copy(
            o_v[b], out_hbm.at[pl.ds(base + c * CHUNK, CHUNK)], sem_o[b])
      return carry

    lax.fori_loop(0, n_chunks // 2, pair_body, 0)
    wait_out(0)
    wait_out(1)

  return lookup


def _prep_body(eye32_ref, eye5_ref, wt_ref, at_ref, w4_ref, a8_ref):
  yt = lax.dot_general(wt_ref[...], eye32_ref[...], (((0,), (0,)), ((), ())),
                       preferred_element_type=jnp.float32)
  yt = yt.reshape(-1, 4, D)
  for k in range(4):
    w4_ref[:, pl.ds(D * k, D)] = yt[:, k, :]
  za = lax.dot_general(at_ref[...], eye5_ref[...], (((0,), (0,)), ((), ())),
                       preferred_element_type=jnp.float32)
  za = za.reshape(-1, 8, R)
  for k in range(8):
    a8_ref[:, pl.ds(RP * k, R)] = za[:, k, :]


def _tc_prep(Wt, At):
  V = Wt.shape[1]
  blk = 16384
  grid = (V + blk - 1) // blk
  return pl.pallas_call(
      _prep_body,
      grid=(grid,),
      in_specs=[
          pl.BlockSpec((D, D), lambda i: (0, 0)),
          pl.BlockSpec((R, R), lambda i: (0, 0)),
          pl.BlockSpec((D, blk), lambda i: (0, i)),
          pl.BlockSpec((R, blk), lambda i: (0, i)),
      ],
      out_specs=[
          pl.BlockSpec((blk // 4, 128), lambda i: (i, 0)),
          pl.BlockSpec((blk // 8, 128), lambda i: (i, 0)),
      ],
      out_shape=[
          jax.ShapeDtypeStruct((V // 4, 128), jnp.float32),
          jax.ShapeDtypeStruct((V // 8, 128), jnp.float32),
      ],
  )(jnp.eye(D, dtype=jnp.float32), jnp.eye(R, dtype=jnp.float32), Wt, At)


def kernel(input, W, A, B):
  orig_shape = input.shape
  idx = input.reshape(-1).astype(jnp.int32)
  n_idx = idx.shape[0]
  V = W.shape[0]
  W4, A8 = _tc_prep(W.T, A.T)
  w_sc = W4.reshape(V, D)
  a_sc = A8.reshape(V, RP)
  out = _make_sc_lookup(n_idx)(idx, w_sc, a_sc, B)
  return out.reshape(*orig_shape, D)


# speedup vs baseline: 1.0219x; 1.0219x over previous
import functools

import jax
import jax.numpy as jnp
from jax import lax
from jax.experimental import pallas as pl
from jax.experimental.pallas import tpu as pltpu
from jax.experimental.pallas import tpu_sc as plsc

D = 32
R = 5
RP = 16
LANES = 16
NUM_WORKERS = 32
CHUNK = 512


def _make_sc_lookup(n_idx: int, n_l: int):
  per_w = n_idx // NUM_WORKERS
  n_chunks = per_w // CHUNK
  n_b = n_idx // n_l
  assert n_chunks % 2 == 0 and n_chunks >= 4
  mesh = plsc.VectorSubcoreMesh(core_axis_name="c", subcore_axis_name="s")

  @functools.partial(
      pl.kernel,
      out_type=jax.ShapeDtypeStruct((n_l, 4, n_b * 8), jnp.float32),
      mesh=mesh,
      scratch_types=[
          [pltpu.VMEM((CHUNK,), jnp.int32) for _ in range(2)],
          [pltpu.VMEM((CHUNK, D), jnp.float32) for _ in range(2)],
          [pltpu.VMEM((CHUNK, RP), jnp.float32) for _ in range(2)],
          [pltpu.VMEM((4, CHUNK * 8), jnp.float32) for _ in range(2)],
          pltpu.VMEM((R, D), jnp.float32),
          [pltpu.SemaphoreType.DMA for _ in range(2)],
          [pltpu.SemaphoreType.DMA for _ in range(2)],
          [pltpu.SemaphoreType.DMA for _ in range(2)],
      ],
      compiler_params=pltpu.CompilerParams(
          use_tc_tiling_on_sc=False, needs_layout_passes=False),
  )
  def lookup(idx_hbm, w_hbm, a_hbm, b_hbm, out_hbm,
             idx_v, w_v, a_v, o_v, b_v, sem_w, sem_a, sem_o):
    wid = lax.axis_index("s") * 2 + lax.axis_index("c")
    base = wid * per_w

    pltpu.sync_copy(b_hbm, b_v)
    b_lo = [b_v[r, pl.ds(0, LANES)] for r in range(R)]
    b_hi = [b_v[r, pl.ds(LANES, LANES)] for r in range(R)]
    lane_r = [jnp.full((LANES, 1), r, jnp.int32) for r in range(R)]
    gdn = lax.GatherDimensionNumbers(
        offset_dims=(), collapsed_slice_dims=(0,), start_index_map=(0,))

    def splat(vec, idx):
      return lax.gather(vec, idx, gdn, (1,),
                        mode=lax.GatherScatterMode.PROMISE_IN_BOUNDS)

    lanes = lax.iota(jnp.int32, LANES)
    dt_lo = lax.shift_right_logical(lanes, 3)
    dt_hi = dt_lo + 2
    dr128 = lax.shift_left(jnp.bitwise_and(lanes, 7), 7)

    def fetch(c, b):
      pltpu.sync_copy(idx_hbm.at[pl.ds(base + c * CHUNK, CHUNK)], idx_v[b])
      pltpu.async_copy(w_hbm.at[idx_v[b]], w_v[b], sem_w[b])
      pltpu.async_copy(a_hbm.at[idx_v[b]], a_v[b], sem_a[b])

    def wait_gathers(b):
      pltpu.make_async_copy(w_hbm.at[pl.ds(0, CHUNK)], w_v[b], sem_w[b]).wait()
      pltpu.make_async_copy(a_hbm.at[pl.ds(0, CHUNK)], a_v[b], sem_a[b]).wait()

    def wait_out(b):
      for _ in range(4):
        pltpu.make_async_copy(
            o_v[b].at[0], out_hbm.at[0, 0, pl.ds(0, CHUNK * 8)],
            sem_o[b]).wait()

    fetch(0, 0)

    def pair_body(gp, carry):
      for b in (0, 1):
        c = gp * 2 + b

        @pl.when(c + 1 < n_chunks)
        def _():
          fetch(c + 1, 1 - b)

        wait_gathers(b)

        @pl.when(c >= 2)
        def _():
          wait_out(b)

        def row_body(i, carry2):
          av = a_v[b][i, pl.ds(0, LANES)]
          lo = w_v[b][i, pl.ds(0, LANES)]
          hi = w_v[b][i, pl.ds(LANES, LANES)]
          for r in range(R):
            a_r = splat(av, lane_r[r])
            lo = lo + a_r * b_lo[r]
            hi = hi + a_r * b_hi[r]
          flat = lax.shift_left(lax.shift_right_logical(i, 7), 10) + \
              jnp.bitwise_and(i, 127)
          flat_v = dr128 + jnp.full((LANES,), flat, jnp.int32)
          plsc.store_scatter(o_v[b], [dt_lo, flat_v], lo)
          plsc.store_scatter(o_v[b], [dt_hi, flat_v], hi)
          return carry2

        lax.fori_loop(0, CHUNK, row_body, 0, unroll=4)
        off = base + c * CHUNK
        l_id = lax.shift_right_logical(off, 12)
        seg = pl.multiple_of(
            lax.shift_left(jnp.bitwise_and(off, n_b - 1), 3), CHUNK * 8)
        for dt in range(4):
          pltpu.async_copy(
              o_v[b].at[dt], out_hbm.at[l_id, dt, pl.ds(seg, CHUNK * 8)],
              sem_o[b])
      return carry

    lax.fori_loop(0, n_chunks // 2, pair_body, 0)
    wait_out(0)
    wait_out(1)

  return lookup


def _prep_body(eye32_ref, eye5_ref, wt_ref, at_ref, w4_ref, a8_ref):
  yt = lax.dot_general(wt_ref[...], eye32_ref[...], (((0,), (0,)), ((), ())),
                       preferred_element_type=jnp.float32)
  yt = yt.reshape(-1, 4, D)
  for k in range(4):
    w4_ref[:, pl.ds(D * k, D)] = yt[:, k, :]
  za = lax.dot_general(at_ref[...], eye5_ref[...], (((0,), (0,)), ((), ())),
                       preferred_element_type=jnp.float32)
  za = za.reshape(-1, 8, R)
  for k in range(8):
    a8_ref[:, pl.ds(RP * k, R)] = za[:, k, :]


def _tc_prep(Wt, At):
  V = Wt.shape[1]
  blk = 16384
  grid = (V + blk - 1) // blk
  return pl.pallas_call(
      _prep_body,
      grid=(grid,),
      in_specs=[
          pl.BlockSpec((D, D), lambda i: (0, 0)),
          pl.BlockSpec((R, R), lambda i: (0, 0)),
          pl.BlockSpec((D, blk), lambda i: (0, i)),
          pl.BlockSpec((R, blk), lambda i: (0, i)),
      ],
      out_specs=[
          pl.BlockSpec((blk // 4, 128), lambda i: (i, 0)),
          pl.BlockSpec((blk // 8, 128), lambda i: (i, 0)),
      ],
      out_shape=[
          jax.ShapeDtypeStruct((V // 4, 128), jnp.float32),
          jax.ShapeDtypeStruct((V // 8, 128), jnp.float32),
      ],
  )(jnp.eye(D, dtype=jnp.float32), jnp.eye(R, dtype=jnp.float32), Wt, At)


def kernel(input, W, A, B):
  n_b, n_l = input.shape
  assert n_b % 128 == 0 and (n_b & (n_b - 1)) == 0
  idx = input.T.reshape(-1).astype(jnp.int32)
  n_idx = idx.shape[0]
  V = W.shape[0]
  W4, A8 = _tc_prep(W.T, A.T)
  w_sc = W4.reshape(V, D)
  a_sc = A8.reshape(V, RP)
  out3 = _make_sc_lookup(n_idx, n_l)(idx, w_sc, a_sc, B)
  out5 = out3.reshape(n_l, 4, n_b // 128, 8, 128)
  return out5.transpose(2, 4, 0, 1, 3).reshape(n_b, n_l, D)
